# baseline (speedup 1.0000x reference)
import functools

import jax
import jax.numpy as jnp
from jax.experimental import pallas as pl
from jax.experimental.pallas import tpu as pltpu

N = 50000
E = 800000
B = 8
HID = 128
HEADS = 4
DH = HID // HEADS


def _ln(h, g, b):
    m = jnp.mean(h, axis=-1, keepdims=True)
    v = jnp.var(h, axis=-1, keepdims=True)
    return (h - m) / jnp.sqrt(v + 1e-5) * g + b


def _gelu(h):
    return jax.nn.gelu(h, approximate=False)


def _pgelu(h):
    return 0.5 * h * (1.0 + jax.lax.erf(h * 0.7071067811865476))


def _softplus(h):
    return jnp.logaddexp(h, 0.0)


def _gat(h, ef, src, dst, p, pre):
    n = h.shape[0]
    xl = (h @ p[pre + '_Wl'] + p[pre + '_bl']).reshape(n, HEADS, DH)
    xr = (h @ p[pre + '_Wr'] + p[pre + '_br']).reshape(n, HEADS, DH)
    ee = (ef @ p[pre + '_We']).reshape(-1, HEADS, DH)
    cnt = jax.ops.segment_sum(jnp.ones((ee.shape[0],), jnp.float32), dst, n)
    loop = jax.ops.segment_sum(ee, dst, n) / jnp.maximum(cnt, 1.0)[:, None, None]
    ar = jnp.arange(n, dtype=src.dtype)
    ms = jnp.concatenate([src, ar])
    md = jnp.concatenate([dst, ar])
    eall = jnp.concatenate([ee, loop], axis=0)
    xj = xl[ms]
    xi = xr[md]
    z = xj + xi + eall
    a = jnp.sum(jax.nn.leaky_relu(z, 0.2) * p[pre + '_att'], axis=-1)
    amax = jax.ops.segment_max(a, md, n)
    amax = jnp.where(jnp.isfinite(amax), amax, 0.0)
    ex = jnp.exp(a - amax[md])
    den = jax.ops.segment_sum(ex, md, n)
    alpha = ex / (den[md] + 1e-16)
    out = jax.ops.segment_sum(xj * alpha[:, :, None], md, n).reshape(n, HID) + p[pre + '_bias']
    return out


def _head_kernel(comb_ref, p1w, p1b, p2w, p2b, p3w, p3b, u1w, u1b, u2w, u2b,
                 preds_ref, unc_ref):
    comb = comb_ref[...]
    h1 = _pgelu(comb @ p1w[...] + p1b[...])
    h2 = _pgelu(h1 @ p2w[...] + p2b[...])
    preds_ref[...] = _softplus(h2 @ p3w[...] + p3b[...])
    u1 = _pgelu(comb @ u1w[...] + u1b[...])
    unc_ref[...] = _softplus(u1 @ u2w[...] + u2b[...])


def kernel(x, edge_index, edge_attr, batch, global_features, params):
    p = params
    src, dst = edge_index[0], edge_index[1]
    h = _gelu(_ln(x @ p['ne_W'] + p['ne_b'], p['ne_g'], p['ne_beta']))
    e = edge_attr @ p['ee_W'] + p['ee_b']
    for i in range(1, 4):
        hres = h
        h = _gat(h, e, src, dst, p, 'c%d' % i)
        h = _ln(h + hres, p['n%d_g' % i], p['n%d_b' % i])
        h = _gelu(h)
    cnt = jax.ops.segment_sum(jnp.ones((N,), jnp.float32), batch, B)
    gm = jax.ops.segment_sum(h, batch, B) / jnp.maximum(cnt, 1.0)[:, None]
    g = _gelu(_ln(global_features @ p['ge_W'] + p['ge_b'], p['ge_g'], p['ge_beta']))
    comb = jnp.concatenate([gm, g], axis=-1)
    preds, unc = pl.pallas_call(
        _head_kernel,
        out_shape=(
            jax.ShapeDtypeStruct((B, 3), jnp.float32),
            jax.ShapeDtypeStruct((B, 3), jnp.float32),
        ),
    )(comb, p['p1_W'], p['p1_b'], p['p2_W'], p['p2_b'], p['p3_W'], p['p3_b'],
      p['u1_W'], p['u1_b'], p['u2_W'], p['u2_b'])
    return preds, unc


# speedup vs baseline: 23.7938x; 23.7938x over previous
import functools

import jax
import jax.numpy as jnp
from jax import lax
from jax.experimental import pallas as pl
from jax.experimental.pallas import tpu as pltpu
from jax.experimental.pallas import tpu_sc as plsc

N = 50000
E = 800000
B = 8
HID = 128
HEADS = 4
DH = HID // HEADS

NC = 2
NS = 16
BLK = 320
EPC = E // NC
BPC = EPC // BLK
NBF = BPC // NS
NBR = BPC - NBF * NS
NPAD = 50048
NT = NPAD // NS
R = 1000
GRID = N // R

_SC_MESH = plsc.VectorSubcoreMesh(core_axis_name="c", subcore_axis_name="s")
_SC_PARAMS = pltpu.CompilerParams(needs_layout_passes=False,
                                  use_tc_tiling_on_sc=False)


def _pgelu(h):
    return 0.5 * h * (1.0 + jax.lax.erf(h * 0.7071067811865476))


def _softplus(h):
    return jnp.logaddexp(h, 0.0)


def _pln(h, g, b):
    m = jnp.mean(h, axis=-1, keepdims=True)
    v = jnp.mean((h - m) ** 2, axis=-1, keepdims=True)
    return (h - m) / jnp.sqrt(v + 1e-5) * g + b


def _dot(a, b):
    return jnp.dot(a, b, preferred_element_type=jnp.float32,
                   precision=jax.lax.Precision.HIGHEST)


def _tile_layout():
    c = lax.axis_index("c")
    s = lax.axis_index("s")
    nblk = jnp.where(s < NBR, NBF + 1, NBF)
    return c, s, nblk


def _blk_base(c, s, j):
    return c * EPC + (s + NS * j) * BLK


@functools.partial(
    pl.kernel,
    out_type=[jax.ShapeDtypeStruct((NC, NPAD, 32), jnp.float32)],
    scratch_types=[
        pltpu.VMEM((BLK,), jnp.int32),
        pltpu.VMEM((BLK, 4), jnp.float32),
        pltpu.VMEM((BLK, 32), jnp.float32),
        pltpu.VMEM_SHARED((NPAD, 32), jnp.float32),
        pltpu.SemaphoreType.DMA,
    ],
    mesh=_SC_MESH,
    compiler_params=_SC_PARAMS,
)
def _sc_deg(dst_hbm, ea4_hbm, z32_hbm, dacc_hbm, dstv, eab, vb, accS, sem):
    c, s, nblk = _tile_layout()
    pltpu.sync_copy(z32_hbm, accS.at[pl.ds(s * NT, NT), :])
    iota = lax.iota(jnp.int32, 16)

    def zrow(r, carry):
        vb[r, pl.ds(0, 16)] = jnp.zeros((16,), jnp.float32)
        vb[r, pl.ds(16, 16)] = jnp.zeros((16,), jnp.float32)
        return carry

    lax.fori_loop(0, BLK, zrow, 0, unroll=4)
    for g in range(BLK // 16):
        plsc.store_scatter(vb, [g * 16 + iota, jnp.full((16,), 3, jnp.int32)],
                           jnp.full((16,), 1.0, jnp.float32))
    plsc.subcore_barrier()

    def body(j, carry):
        base = _blk_base(c, s, j)
        pltpu.sync_copy(dst_hbm.at[pl.ds(base, BLK)], dstv)
        pltpu.sync_copy(ea4_hbm.at[pl.ds(base, BLK), :], eab)

        def group(g, carry2):
            eidx = g * 16 + iota
            for k in range(3):
                kv = jnp.full((16,), k, jnp.int32)
                plsc.store_scatter(vb, [eidx, kv],
                                   plsc.load_gather(eab, [eidx, kv]))
            return carry2

        lax.fori_loop(0, BLK // 16, group, 0)
        pltpu.sync_copy(vb, accS.at[dstv], add=True)
        return carry

    lax.fori_loop(0, nblk, body, 0)
    plsc.subcore_barrier()
    pltpu.sync_copy(accS.at[pl.ds(s * NT, NT), :],
                    dacc_hbm.at[c, pl.ds(s * NT, NT), :])


@functools.partial(
    pl.kernel,
    out_type=[
        jax.ShapeDtypeStruct((E,), jnp.float32),
        jax.ShapeDtypeStruct((E,), jnp.float32),
        jax.ShapeDtypeStruct((E,), jnp.float32),
        jax.ShapeDtypeStruct((E,), jnp.float32),
        jax.ShapeDtypeStruct((NC, NPAD), jnp.float32),
        jax.ShapeDtypeStruct((NC, NPAD), jnp.float32),
        jax.ShapeDtypeStruct((NC, NPAD), jnp.float32),
        jax.ShapeDtypeStruct((NC, NPAD), jnp.float32),
    ],
    scratch_types=[
        pltpu.VMEM((BLK,), jnp.int32),
        pltpu.VMEM((BLK,), jnp.int32),
        pltpu.VMEM((BLK, 4), jnp.float32),
        pltpu.VMEM((BLK, HID), jnp.float32),
        pltpu.VMEM((BLK, HID), jnp.float32),
        pltpu.VMEM((BLK,), jnp.float32),
        pltpu.VMEM((BLK,), jnp.float32),
        pltpu.VMEM((BLK,), jnp.float32),
        pltpu.VMEM((BLK,), jnp.float32),
        pltpu.VMEM((4, HID), jnp.float32),
        pltpu.VMEM((HID,), jnp.float32),
        pltpu.VMEM_SHARED((NPAD,), jnp.float32),
        pltpu.VMEM_SHARED((NPAD,), jnp.float32),
        pltpu.VMEM_SHARED((NPAD,), jnp.float32),
        pltpu.VMEM_SHARED((NPAD,), jnp.float32),
        pltpu.SemaphoreType.DMA,
        pltpu.SemaphoreType.DMA,
    ],
    mesh=_SC_MESH,
    compiler_params=_SC_PARAMS,
)
def _sc_logits(xl_hbm, xrf_hbm, src_hbm, dst_hbm, ea4_hbm, wip_hbm, att_hbm,
               z1_hbm, ex0_hbm, ex1_hbm, ex2_hbm, ex3_hbm,
               den0_hbm, den1_hbm, den2_hbm, den3_hbm,
               srcv, dstv, eab, xlb, xrb, exb0, exb1, exb2, exb3,
               wipv, attb, acc0, acc1, acc2, acc3, sem, sem2):
    c, s, nblk = _tile_layout()
    accs = [acc0, acc1, acc2, acc3]
    dens = [den0_hbm, den1_hbm, den2_hbm, den3_hbm]
    exbs = [exb0, exb1, exb2, exb3]
    pltpu.sync_copy(wip_hbm, wipv)
    pltpu.sync_copy(att_hbm, attb)
    for a in accs:
        pltpu.sync_copy(z1_hbm, a.at[pl.ds(s * NT, NT)])
    plsc.subcore_barrier()
    iota = lax.iota(jnp.int32, 16)

    def body(j, carry):
        base = _blk_base(c, s, j)
        pltpu.sync_copy(src_hbm.at[pl.ds(base, BLK)], srcv)
        pltpu.sync_copy(dst_hbm.at[pl.ds(base, BLK)], dstv)
        pltpu.sync_copy(ea4_hbm.at[pl.ds(base, BLK), :], eab)
        pltpu.async_copy(xl_hbm.at[srcv], xlb, sem).wait()
        pltpu.async_copy(xrf_hbm.at[dstv], xrb, sem2).wait()

        def group(g, carry2):
            eidx = g * 16 + iota
            ea0 = plsc.load_gather(eab, [eidx, jnp.zeros((16,), jnp.int32)])
            ea1 = plsc.load_gather(eab, [eidx, jnp.full((16,), 1, jnp.int32)])
            ea2 = plsc.load_gather(eab, [eidx, jnp.full((16,), 2, jnp.int32)])
            zero16 = jnp.zeros((16,), jnp.int32)
            for h in range(HEADS):
                a_h = jnp.zeros((16,), jnp.float32)
                for cc in range(h * DH, (h + 1) * DH):
                    cv = jnp.full((16,), cc, jnp.int32)
                    w0 = plsc.load_gather(wipv, [zero16, cv])
                    w1 = plsc.load_gather(wipv, [zero16 + 1, cv])
                    w2 = plsc.load_gather(wipv, [zero16 + 2, cv])
                    av = plsc.load_gather(attb, [cv])
                    z = (plsc.load_gather(xlb, [eidx, cv])
                         + plsc.load_gather(xrb, [eidx, cv])
                         + ea0 * w0 + ea1 * w1 + ea2 * w2)
                    lz = jnp.maximum(z, 0.0) + 0.2 * jnp.minimum(z, 0.0)
                    a_h = a_h + lz * av
                ex_h = jnp.exp(a_h)
                [exb0, exb1, exb2, exb3][h][pl.ds(g * 16, 16)] = ex_h
            return carry2

        lax.fori_loop(0, BLK // 16, group, 0)
        for h, (exb, exh) in enumerate(
                zip(exbs, [ex0_hbm, ex1_hbm, ex2_hbm, ex3_hbm])):
            pltpu.sync_copy(exb, accs[h].at[dstv], add=True)
            pltpu.sync_copy(exb, exh.at[pl.ds(base, BLK)])
        return carry

    lax.fori_loop(0, nblk, body, 0)
    plsc.subcore_barrier()
    for h in range(HEADS):
        pltpu.sync_copy(accs[h].at[pl.ds(s * NT, NT)],
                        dens[h].at[c, pl.ds(s * NT, NT)])


@functools.partial(
    pl.kernel,
    out_type=[jax.ShapeDtypeStruct((NC, HEADS, NPAD, DH), jnp.float32)],
    scratch_types=[
        pltpu.VMEM((BLK,), jnp.int32),
        pltpu.VMEM((BLK,), jnp.int32),
        pltpu.VMEM((BLK,), jnp.float32),
        pltpu.VMEM((BLK, DH), jnp.float32),
        pltpu.VMEM((BLK, DH), jnp.float32),
        pltpu.VMEM_SHARED((NPAD, DH), jnp.float32),
        pltpu.SemaphoreType.DMA,
    ],
    mesh=_SC_MESH,
    compiler_params=_SC_PARAMS,
)
def _sc_scatter(xlh0_hbm, xlh1_hbm, xlh2_hbm, xlh3_hbm, src_hbm, dst_hbm,
                ex0_hbm, ex1_hbm, ex2_hbm, ex3_hbm, z32_hbm, onum_hbm,
                srcv, dstv, wv, xb, ob, accS, sem):
    c, s, nblk = _tile_layout()
    xlh = [xlh0_hbm, xlh1_hbm, xlh2_hbm, xlh3_hbm]
    exa = [ex0_hbm, ex1_hbm, ex2_hbm, ex3_hbm]
    for h in range(HEADS):
        pltpu.sync_copy(z32_hbm, accS.at[pl.ds(s * NT, NT), :])
        plsc.subcore_barrier()

        def body(j, carry):
            base = _blk_base(c, s, j)
            pltpu.sync_copy(src_hbm.at[pl.ds(base, BLK)], srcv)
            pltpu.sync_copy(dst_hbm.at[pl.ds(base, BLK)], dstv)
            pltpu.sync_copy(exa[h].at[pl.ds(base, BLK)], wv)
            pltpu.async_copy(xlh[h].at[srcv], xb, sem).wait()

            def edge(e, carry2):
                w = plsc.load_gather(wv, [jnp.full((16,), e, jnp.int32)])
                ob[e, pl.ds(0, 16)] = xb[e, pl.ds(0, 16)] * w
                ob[e, pl.ds(16, 16)] = xb[e, pl.ds(16, 16)] * w
                return carry2

            lax.fori_loop(0, BLK, edge, 0, unroll=4)
            pltpu.sync_copy(ob, accS.at[dstv], add=True)
            return carry

        lax.fori_loop(0, nblk, body, 0)
        plsc.subcore_barrier()
        pltpu.sync_copy(accS.at[pl.ds(s * NT, NT), :],
                        onum_hbm.at[c, h, pl.ds(s * NT, NT), :])
        plsc.subcore_barrier()


def _node_enc_kernel(x_ref, w_ref, b_ref, g_ref, beta_ref, o_ref):
    h = _dot(x_ref[...], w_ref[...]) + b_ref[...]
    o_ref[...] = _pgelu(_pln(h, g_ref[...], beta_ref[...]))


def _prep_kernel(h_ref, a0_ref, a1_ref, wl_ref, bl_ref, wr_ref, br_ref,
                 wip_ref, bip_ref, att_ref, sel_ref,
                 xl_ref, xrf_ref, exs_ref, x0_ref, x1_ref, x2_ref, x3_ref):
    h = h_ref[...]
    xl = _dot(h, wl_ref[...]) + bl_ref[...]
    xr = _dot(h, wr_ref[...]) + br_ref[...]
    bip = bip_ref[...]
    xl_ref[...] = xl
    xrf_ref[...] = xr + bip
    acc = a0_ref[...] + a1_ref[...]
    cnt = acc[:, 3:4]
    mc = jnp.maximum(cnt, 1.0)
    ea_mean = acc[:, 0:3] / mc
    scale = cnt / mc
    loop = _dot(ea_mean, wip_ref[...]) + scale * bip
    zs = xl + xr + loop
    lz = (jnp.maximum(zs, 0.0) + 0.2 * jnp.minimum(zs, 0.0)) * att_ref[...]
    exs_ref[...] = jnp.exp(_dot(lz, sel_ref[...]))
    x0_ref[...] = xl[:, 0:32]
    x1_ref[...] = xl[:, 32:64]
    x2_ref[...] = xl[:, 64:96]
    x3_ref[...] = xl[:, 96:128]


def _combine_kernel(p00, p01, p02, p03, p10, p11, p12, p13, den_ref,
                    exs_ref, xl_ref, hres_ref, bias_ref, g_ref, b_ref,
                    ex_ref, o_ref):
    exs = exs_ref[...]
    den = den_ref[...] + exs
    den128 = _dot(den, ex_ref[...])
    exs128 = _dot(exs, ex_ref[...])
    xl = xl_ref[...]
    num = jnp.concatenate(
        [p00[...] + p10[...], p01[...] + p11[...],
         p02[...] + p12[...], p03[...] + p13[...]], axis=-1)
    num = num + exs128 * xl
    out = num / den128 + bias_ref[...] + hres_ref[...]
    o_ref[...] = _pgelu(_pln(out, g_ref[...], b_ref[...]))


def _pool_kernel(h_ref, bat_ref, gm_ref, acc_ref, cnt_ref):
    i = pl.program_id(0)

    @pl.when(i == 0)
    def _():
        acc_ref[...] = jnp.zeros_like(acc_ref)
        cnt_ref[...] = jnp.zeros_like(cnt_ref)

    h = h_ref[...]
    bat = bat_ref[...]
    for b in range(B):
        mask = jnp.where(bat == b, 1.0, 0.0)
        acc_ref[b, :] += jnp.sum(mask * h, axis=0)
        cnt_ref[b, :] += jnp.sum(mask, axis=0) * jnp.ones((HID,), jnp.float32)

    @pl.when(i == GRID - 1)
    def _():
        gm_ref[...] = acc_ref[...] / jnp.maximum(cnt_ref[...], 1.0)


def _head_kernel(gm_ref, gfp_ref, gew_ref, geb_ref, geg_ref, gebeta_ref,
                 p1a, p1b_w, p1b, p2w, p2b, p3w, p3b,
                 u1a, u1b_w, u1b, u2w, u2b, preds_ref, unc_ref):
    gm = gm_ref[...]
    ge = _dot(gfp_ref[...], gew_ref[...]) + geb_ref[...]
    g = _pgelu(_pln(ge, geg_ref[...], gebeta_ref[...]))
    h1 = _pgelu(_dot(gm, p1a[...]) + _dot(g, p1b_w[...]) + p1b[...])
    h2 = _pgelu(_dot(h1, p2w[...]) + p2b[...])
    preds_ref[...] = _softplus(_dot(h2, p3w[...]) + p3b[...])
    u1 = _pgelu(_dot(gm, u1a[...]) + _dot(g, u1b_w[...]) + u1b[...])
    unc_ref[...] = _softplus(_dot(u1, u2w[...]) + u2b[...])


def _row_spec(cols):
    return pl.BlockSpec((R, cols), lambda i: (i, 0))


def _full_spec(shape):
    nd = len(shape)
    return pl.BlockSpec(shape, lambda i: (0,) * nd)


def kernel(x, edge_index, edge_attr, batch, global_features, params):
    p = params
    src, dst = edge_index[0], edge_index[1]
    ea4 = jnp.concatenate([edge_attr, jnp.ones((E, 1), jnp.float32)], axis=1)
    z32 = jnp.zeros((NT, 32), jnp.float32)
    z1 = jnp.zeros((NT,), jnp.float32)
    sel = jnp.repeat(jnp.eye(HEADS, dtype=jnp.float32), DH, axis=0)

    h = pl.pallas_call(
        _node_enc_kernel,
        grid=(GRID,),
        in_specs=[_row_spec(4), _full_spec((4, HID)), _full_spec((HID,)),
                  _full_spec((HID,)), _full_spec((HID,))],
        out_specs=_row_spec(HID),
        out_shape=jax.ShapeDtypeStruct((N, HID), jnp.float32),
    )(x, p['ne_W'], p['ne_b'], p['ne_g'], p['ne_beta'])

    dacc = _sc_deg(dst, ea4, z32)[0]

    for i in range(1, 4):
        pre = 'c%d' % i
        wip = p['ee_W'] @ p[pre + '_We']
        bip = p['ee_b'] @ p[pre + '_We']
        wipp = jnp.concatenate([wip, jnp.zeros((1, HID), jnp.float32)], 0)
        attf = p[pre + '_att'].reshape(HID)
        hres = h

        xl, xrf, exs, x0, x1, x2, x3 = pl.pallas_call(
            _prep_kernel,
            grid=(GRID,),
            in_specs=[_row_spec(HID), _row_spec(32), _row_spec(32),
                      _full_spec((HID, HID)), _full_spec((HID,)),
                      _full_spec((HID, HID)), _full_spec((HID,)),
                      _full_spec((3, HID)), _full_spec((HID,)),
                      _full_spec((HID,)), _full_spec((HID, HEADS))],
            out_specs=[_row_spec(HID), _row_spec(HID), _row_spec(4),
                       _row_spec(DH), _row_spec(DH), _row_spec(DH),
                       _row_spec(DH)],
            out_shape=[jax.ShapeDtypeStruct((N, HID), jnp.float32),
                       jax.ShapeDtypeStruct((N, HID), jnp.float32),
                       jax.ShapeDtypeStruct((N, HEADS), jnp.float32),
                       jax.ShapeDtypeStruct((N, DH), jnp.float32),
                       jax.ShapeDtypeStruct((N, DH), jnp.float32),
                       jax.ShapeDtypeStruct((N, DH), jnp.float32),
                       jax.ShapeDtypeStruct((N, DH), jnp.float32)],
        )(h, dacc[0], dacc[1], p[pre + '_Wl'], p[pre + '_bl'],
          p[pre + '_Wr'], p[pre + '_br'], wip, bip, attf, sel)

        ex0, ex1, ex2, ex3, dn0, dn1, dn2, dn3 = _sc_logits(
            xl, xrf, src, dst, ea4, wipp, attf, z1)
        den4 = jnp.stack([dn0[0] + dn0[1], dn1[0] + dn1[1],
                          dn2[0] + dn2[1], dn3[0] + dn3[1]], axis=1)
        onum = _sc_scatter(x0, x1, x2, x3, src, dst,
                           ex0, ex1, ex2, ex3, z32[:, :DH])[0]

        h = pl.pallas_call(
            _combine_kernel,
            grid=(GRID,),
            in_specs=[_row_spec(DH)] * 8 + [_row_spec(4),
                      _row_spec(4), _row_spec(HID), _row_spec(HID),
                      _full_spec((HID,)), _full_spec((HID,)),
                      _full_spec((HID,)), _full_spec((HEADS, HID))],
            out_specs=_row_spec(HID),
            out_shape=jax.ShapeDtypeStruct((N, HID), jnp.float32),
        )(onum[0, 0], onum[0, 1], onum[0, 2], onum[0, 3],
          onum[1, 0], onum[1, 1], onum[1, 2], onum[1, 3],
          den4, exs, xl, hres, p[pre + '_bias'],
          p['n%d_g' % i], p['n%d_b' % i], sel.T)

    gm = pl.pallas_call(
        _pool_kernel,
        grid=(GRID,),
        in_specs=[_row_spec(HID), pl.BlockSpec((R, 1), lambda i: (i, 0))],
        out_specs=_full_spec((B, HID)),
        out_shape=jax.ShapeDtypeStruct((B, HID), jnp.float32),
        scratch_shapes=[pltpu.VMEM((B, HID), jnp.float32),
                        pltpu.VMEM((B, HID), jnp.float32)],
    )(h, batch.reshape(N, 1))

    gfp = jnp.pad(global_features, ((0, 0), (0, 5)))
    gewp = jnp.pad(p['ge_W'], ((0, 5), (0, 0)))
    preds, unc = pl.pallas_call(
        _head_kernel,
        out_shape=(jax.ShapeDtypeStruct((B, 3), jnp.float32),
                   jax.ShapeDtypeStruct((B, 3), jnp.float32)),
    )(gm, gfp, gewp, p['ge_b'], p['ge_g'], p['ge_beta'],
      p['p1_W'][:HID], p['p1_W'][HID:], p['p1_b'], p['p2_W'], p['p2_b'],
      p['p3_W'], p['p3_b'],
      p['u1_W'][:HID], p['u1_W'][HID:], p['u1_b'], p['u2_W'], p['u2_b'])
    return preds, unc


# speedup vs baseline: 24.1539x; 1.0151x over previous
import functools

import jax
import jax.numpy as jnp
from jax import lax
from jax.experimental import pallas as pl
from jax.experimental.pallas import tpu as pltpu
from jax.experimental.pallas import tpu_sc as plsc

N = 50000
E = 800000
B = 8
HID = 128
HEADS = 4
DH = HID // HEADS

NC = 2
NS = 16
BLK = 320
EPC = E // NC
BPC = EPC // BLK
NBF = BPC // NS
NBR = BPC - NBF * NS
NPAD = 50048
NT = NPAD // NS
R = 1000
GRID = N // R

_SC_MESH = plsc.VectorSubcoreMesh(core_axis_name="c", subcore_axis_name="s")
_SC_PARAMS = pltpu.CompilerParams(needs_layout_passes=False,
                                  use_tc_tiling_on_sc=False)


def _pgelu(h):
    return 0.5 * h * (1.0 + jax.lax.erf(h * 0.7071067811865476))


def _softplus(h):
    return jnp.logaddexp(h, 0.0)


def _pln(h, g, b):
    m = jnp.mean(h, axis=-1, keepdims=True)
    v = jnp.mean((h - m) ** 2, axis=-1, keepdims=True)
    return (h - m) / jnp.sqrt(v + 1e-5) * g + b


def _dot(a, b):
    return jnp.dot(a, b, preferred_element_type=jnp.float32,
                   precision=jax.lax.Precision.HIGHEST)


def _tile_layout():
    c = lax.axis_index("c")
    s = lax.axis_index("s")
    nblk = jnp.where(s < NBR, NBF + 1, NBF)
    return c, s, nblk


def _blk_base(c, s, j):
    return c * EPC + (s + NS * j) * BLK


@functools.partial(
    pl.kernel,
    out_type=[jax.ShapeDtypeStruct((NC, NPAD, 32), jnp.float32)],
    scratch_types=[
        pltpu.VMEM((BLK,), jnp.int32),
        pltpu.VMEM((BLK, 3), jnp.float32),
        pltpu.VMEM((BLK, 32), jnp.float32),
        pltpu.VMEM_SHARED((NPAD, 32), jnp.float32),
        pltpu.SemaphoreType.DMA,
    ],
    mesh=_SC_MESH,
    compiler_params=_SC_PARAMS,
)
def _sc_deg(dst_hbm, ea_hbm, z32_hbm, dacc_hbm, dstv, eab, vb, accS, sem):
    c, s, nblk = _tile_layout()
    pltpu.sync_copy(z32_hbm, accS.at[pl.ds(s * NT, NT), :])
    iota = lax.iota(jnp.int32, 16)

    def zrow(r, carry):
        vb[r, pl.ds(0, 16)] = jnp.zeros((16,), jnp.float32)
        vb[r, pl.ds(16, 16)] = jnp.zeros((16,), jnp.float32)
        return carry

    lax.fori_loop(0, BLK, zrow, 0, unroll=4)
    for g in range(BLK // 16):
        plsc.store_scatter(vb, [g * 16 + iota, jnp.full((16,), 3, jnp.int32)],
                           jnp.full((16,), 1.0, jnp.float32))
    plsc.subcore_barrier()

    def body(j, carry):
        base = _blk_base(c, s, j)
        pltpu.sync_copy(dst_hbm.at[pl.ds(base, BLK)], dstv)
        pltpu.sync_copy(ea_hbm.at[pl.ds(base, BLK), :], eab)

        def group(g, carry2):
            eidx = g * 16 + iota
            for k in range(3):
                kv = jnp.full((16,), k, jnp.int32)
                plsc.store_scatter(vb, [eidx, kv],
                                   plsc.load_gather(eab, [eidx, kv]))
            return carry2

        lax.fori_loop(0, BLK // 16, group, 0)
        pltpu.sync_copy(vb, accS.at[dstv], add=True)
        return carry

    lax.fori_loop(0, nblk, body, 0)
    plsc.subcore_barrier()
    pltpu.sync_copy(accS.at[pl.ds(s * NT, NT), :],
                    dacc_hbm.at[c, pl.ds(s * NT, NT), :])


@functools.partial(
    pl.kernel,
    out_type=[
        jax.ShapeDtypeStruct((E,), jnp.float32),
        jax.ShapeDtypeStruct((E,), jnp.float32),
        jax.ShapeDtypeStruct((E,), jnp.float32),
        jax.ShapeDtypeStruct((E,), jnp.float32),
        jax.ShapeDtypeStruct((NC, NPAD), jnp.float32),
        jax.ShapeDtypeStruct((NC, NPAD), jnp.float32),
        jax.ShapeDtypeStruct((NC, NPAD), jnp.float32),
        jax.ShapeDtypeStruct((NC, NPAD), jnp.float32),
    ],
    scratch_types=[
        pltpu.VMEM((BLK,), jnp.int32),
        pltpu.VMEM((BLK,), jnp.int32),
        pltpu.VMEM((BLK, 3), jnp.float32),
        pltpu.VMEM((BLK, HID), jnp.float32),
        pltpu.VMEM((BLK, HID), jnp.float32),
        pltpu.VMEM((BLK,), jnp.float32),
        pltpu.VMEM((BLK,), jnp.float32),
        pltpu.VMEM((BLK,), jnp.float32),
        pltpu.VMEM((BLK,), jnp.float32),
        pltpu.VMEM((4, HID), jnp.float32),
        pltpu.VMEM((HID,), jnp.float32),
        pltpu.VMEM_SHARED((NPAD,), jnp.float32),
        pltpu.VMEM_SHARED((NPAD,), jnp.float32),
        pltpu.VMEM_SHARED((NPAD,), jnp.float32),
        pltpu.VMEM_SHARED((NPAD,), jnp.float32),
        pltpu.SemaphoreType.DMA,
        pltpu.SemaphoreType.DMA,
    ],
    mesh=_SC_MESH,
    compiler_params=_SC_PARAMS,
)
def _sc_logits(xl_hbm, xrf_hbm, src_hbm, dst_hbm, ea_hbm, wip_hbm, att_hbm,
               z1_hbm, ex0_hbm, ex1_hbm, ex2_hbm, ex3_hbm,
               den0_hbm, den1_hbm, den2_hbm, den3_hbm,
               srcv, dstv, eab, xlb, xrb, exb0, exb1, exb2, exb3,
               wipv, attb, acc0, acc1, acc2, acc3, sem, sem2):
    c, s, nblk = _tile_layout()
    accs = [acc0, acc1, acc2, acc3]
    dens = [den0_hbm, den1_hbm, den2_hbm, den3_hbm]
    exbs = [exb0, exb1, exb2, exb3]
    pltpu.sync_copy(wip_hbm, wipv)
    pltpu.sync_copy(att_hbm, attb)
    for a in accs:
        pltpu.sync_copy(z1_hbm, a.at[pl.ds(s * NT, NT)])
    plsc.subcore_barrier()
    iota = lax.iota(jnp.int32, 16)

    def body(j, carry):
        base = _blk_base(c, s, j)
        pltpu.sync_copy(src_hbm.at[pl.ds(base, BLK)], srcv)
        pltpu.sync_copy(dst_hbm.at[pl.ds(base, BLK)], dstv)
        pltpu.sync_copy(ea_hbm.at[pl.ds(base, BLK), :], eab)
        pltpu.async_copy(xl_hbm.at[srcv], xlb, sem).wait()
        pltpu.async_copy(xrf_hbm.at[dstv], xrb, sem2).wait()

        def group(g, carry2):
            eidx = g * 16 + iota
            ea0 = plsc.load_gather(eab, [eidx, jnp.zeros((16,), jnp.int32)])
            ea1 = plsc.load_gather(eab, [eidx, jnp.full((16,), 1, jnp.int32)])
            ea2 = plsc.load_gather(eab, [eidx, jnp.full((16,), 2, jnp.int32)])
            zero16 = jnp.zeros((16,), jnp.int32)
            for h in range(HEADS):
                a_h = jnp.zeros((16,), jnp.float32)
                for cc in range(h * DH, (h + 1) * DH):
                    cv = jnp.full((16,), cc, jnp.int32)
                    w0 = plsc.load_gather(wipv, [zero16, cv])
                    w1 = plsc.load_gather(wipv, [zero16 + 1, cv])
                    w2 = plsc.load_gather(wipv, [zero16 + 2, cv])
                    av = plsc.load_gather(attb, [cv])
                    z = (plsc.load_gather(xlb, [eidx, cv])
                         + plsc.load_gather(xrb, [eidx, cv])
                         + ea0 * w0 + ea1 * w1 + ea2 * w2)
                    lz = jnp.maximum(z, 0.0) + 0.2 * jnp.minimum(z, 0.0)
                    a_h = a_h + lz * av
                ex_h = jnp.exp(a_h)
                [exb0, exb1, exb2, exb3][h][pl.ds(g * 16, 16)] = ex_h
            return carry2

        lax.fori_loop(0, BLK // 16, group, 0)
        for h, (exb, exh) in enumerate(
                zip(exbs, [ex0_hbm, ex1_hbm, ex2_hbm, ex3_hbm])):
            pltpu.sync_copy(exb, accs[h].at[dstv], add=True)
            pltpu.sync_copy(exb, exh.at[pl.ds(base, BLK)])
        return carry

    lax.fori_loop(0, nblk, body, 0)
    plsc.subcore_barrier()
    for h in range(HEADS):
        pltpu.sync_copy(accs[h].at[pl.ds(s * NT, NT)],
                        dens[h].at[c, pl.ds(s * NT, NT)])


@functools.partial(
    pl.kernel,
    out_type=[jax.ShapeDtypeStruct((NPAD, DH), jnp.float32)] * 8,
    scratch_types=[
        pltpu.VMEM((BLK,), jnp.int32),
        pltpu.VMEM((BLK,), jnp.int32),
        pltpu.VMEM((BLK,), jnp.float32),
        pltpu.VMEM((BLK, DH), jnp.float32),
        pltpu.VMEM((BLK, DH), jnp.float32),
        pltpu.VMEM_SHARED((NPAD, DH), jnp.float32),
        pltpu.SemaphoreType.DMA,
    ],
    mesh=_SC_MESH,
    compiler_params=_SC_PARAMS,
)
def _sc_scatter(xlh0_hbm, xlh1_hbm, xlh2_hbm, xlh3_hbm, src_hbm, dst_hbm,
                ex0_hbm, ex1_hbm, ex2_hbm, ex3_hbm, z32_hbm,
                o00, o01, o02, o03, o10, o11, o12, o13,
                srcv, dstv, wv, xb, ob, accS, sem):
    onum_out = [[o00, o01, o02, o03], [o10, o11, o12, o13]]
    c, s, nblk = _tile_layout()
    xlh = [xlh0_hbm, xlh1_hbm, xlh2_hbm, xlh3_hbm]
    exa = [ex0_hbm, ex1_hbm, ex2_hbm, ex3_hbm]
    for h in range(HEADS):
        pltpu.sync_copy(z32_hbm, accS.at[pl.ds(s * NT, NT), :])
        plsc.subcore_barrier()

        def body(j, carry):
            base = _blk_base(c, s, j)
            pltpu.sync_copy(src_hbm.at[pl.ds(base, BLK)], srcv)
            pltpu.sync_copy(dst_hbm.at[pl.ds(base, BLK)], dstv)
            pltpu.sync_copy(exa[h].at[pl.ds(base, BLK)], wv)
            pltpu.async_copy(xlh[h].at[srcv], xb, sem).wait()

            def edge(e, carry2):
                w = plsc.load_gather(wv, [jnp.full((16,), e, jnp.int32)])
                ob[e, pl.ds(0, 16)] = xb[e, pl.ds(0, 16)] * w
                ob[e, pl.ds(16, 16)] = xb[e, pl.ds(16, 16)] * w
                return carry2

            lax.fori_loop(0, BLK, edge, 0, unroll=4)
            pltpu.sync_copy(ob, accS.at[dstv], add=True)
            return carry

        lax.fori_loop(0, nblk, body, 0)
        plsc.subcore_barrier()

        @pl.when(c == 0)
        def _():
            pltpu.sync_copy(accS.at[pl.ds(s * NT, NT), :],
                            onum_out[0][h].at[pl.ds(s * NT, NT), :])

        @pl.when(c == 1)
        def _():
            pltpu.sync_copy(accS.at[pl.ds(s * NT, NT), :],
                            onum_out[1][h].at[pl.ds(s * NT, NT), :])

        plsc.subcore_barrier()


def _node_enc_kernel(x_ref, w_ref, b_ref, g_ref, beta_ref, o_ref):
    h = _dot(x_ref[...], w_ref[...]) + b_ref[...]
    o_ref[...] = _pgelu(_pln(h, g_ref[...], beta_ref[...]))


def _prep_kernel(h_ref, a0_ref, a1_ref, wl_ref, bl_ref, wr_ref, br_ref,
                 wip_ref, bip_ref, att_ref, sel_ref,
                 xl_ref, xrf_ref, exs_ref, x0_ref, x1_ref, x2_ref, x3_ref):
    h = h_ref[...]
    xl = _dot(h, wl_ref[...]) + bl_ref[...]
    xr = _dot(h, wr_ref[...]) + br_ref[...]
    bip = bip_ref[...]
    xl_ref[...] = xl
    xrf_ref[...] = xr + bip
    acc = a0_ref[...] + a1_ref[...]
    cnt = acc[:, 3:4]
    mc = jnp.maximum(cnt, 1.0)
    ea_mean = acc[:, 0:3] / mc
    scale = cnt / mc
    loop = _dot(ea_mean, wip_ref[...]) + scale * bip
    zs = xl + xr + loop
    lz = (jnp.maximum(zs, 0.0) + 0.2 * jnp.minimum(zs, 0.0)) * att_ref[...]
    exs_ref[...] = jnp.exp(_dot(lz, sel_ref[...]))
    x0_ref[...] = xl[:, 0:32]
    x1_ref[...] = xl[:, 32:64]
    x2_ref[...] = xl[:, 64:96]
    x3_ref[...] = xl[:, 96:128]


def _combine_kernel(p00, p01, p02, p03, p10, p11, p12, p13, den_ref,
                    exs_ref, xl_ref, hres_ref, bias_ref, g_ref, b_ref,
                    ex_ref, o_ref):
    exs = exs_ref[...]
    den = den_ref[...] + exs
    den128 = _dot(den, ex_ref[...])
    exs128 = _dot(exs, ex_ref[...])
    xl = xl_ref[...]
    num = jnp.concatenate(
        [p00[...] + p10[...], p01[...] + p11[...],
         p02[...] + p12[...], p03[...] + p13[...]], axis=-1)
    num = num + exs128 * xl
    out = num / den128 + bias_ref[...] + hres_ref[...]
    o_ref[...] = _pgelu(_pln(out, g_ref[...], b_ref[...]))


def _pool_kernel(h_ref, bat_ref, gm_ref, acc_ref, cnt_ref):
    i = pl.program_id(0)

    @pl.when(i == 0)
    def _():
        acc_ref[...] = jnp.zeros_like(acc_ref)
        cnt_ref[...] = jnp.zeros_like(cnt_ref)

    h = h_ref[...]
    bat = bat_ref[...]
    for b in range(B):
        mask = jnp.where(bat == b, 1.0, 0.0)
        acc_ref[b, :] += jnp.sum(mask * h, axis=0)
        cnt_ref[b, :] += jnp.sum(mask, axis=0) * jnp.ones((HID,), jnp.float32)

    @pl.when(i == GRID - 1)
    def _():
        gm_ref[...] = acc_ref[...] / jnp.maximum(cnt_ref[...], 1.0)


def _head_kernel(gm_ref, gfp_ref, gew_ref, geb_ref, geg_ref, gebeta_ref,
                 p1a, p1b_w, p1b, p2w, p2b, p3w, p3b,
                 u1a, u1b_w, u1b, u2w, u2b, preds_ref, unc_ref):
    gm = gm_ref[...]
    ge = _dot(gfp_ref[...], gew_ref[...]) + geb_ref[...]
    g = _pgelu(_pln(ge, geg_ref[...], gebeta_ref[...]))
    h1 = _pgelu(_dot(gm, p1a[...]) + _dot(g, p1b_w[...]) + p1b[...])
    h2 = _pgelu(_dot(h1, p2w[...]) + p2b[...])
    preds_ref[...] = _softplus(_dot(h2, p3w[...]) + p3b[...])
    u1 = _pgelu(_dot(gm, u1a[...]) + _dot(g, u1b_w[...]) + u1b[...])
    unc_ref[...] = _softplus(_dot(u1, u2w[...]) + u2b[...])


def _row_spec(cols):
    return pl.BlockSpec((R, cols), lambda i: (i, 0))


def _full_spec(shape):
    nd = len(shape)
    return pl.BlockSpec(shape, lambda i: (0,) * nd)


def kernel(x, edge_index, edge_attr, batch, global_features, params):
    p = params
    src, dst = edge_index[0], edge_index[1]
    z32 = jnp.zeros((NT, 32), jnp.float32)
    z1 = jnp.zeros((NT,), jnp.float32)
    sel = jnp.repeat(jnp.eye(HEADS, dtype=jnp.float32), DH, axis=0)

    h = pl.pallas_call(
        _node_enc_kernel,
        grid=(GRID,),
        in_specs=[_row_spec(4), _full_spec((4, HID)), _full_spec((HID,)),
                  _full_spec((HID,)), _full_spec((HID,))],
        out_specs=_row_spec(HID),
        out_shape=jax.ShapeDtypeStruct((N, HID), jnp.float32),
    )(x, p['ne_W'], p['ne_b'], p['ne_g'], p['ne_beta'])

    dacc = _sc_deg(dst, edge_attr, z32)[0]

    for i in range(1, 4):
        pre = 'c%d' % i
        wip = p['ee_W'] @ p[pre + '_We']
        bip = p['ee_b'] @ p[pre + '_We']
        wipp = jnp.concatenate([wip, jnp.zeros((1, HID), jnp.float32)], 0)
        attf = p[pre + '_att'].reshape(HID)
        hres = h

        xl, xrf, exs, x0, x1, x2, x3 = pl.pallas_call(
            _prep_kernel,
            grid=(GRID,),
            in_specs=[_row_spec(HID), _row_spec(32), _row_spec(32),
                      _full_spec((HID, HID)), _full_spec((HID,)),
                      _full_spec((HID, HID)), _full_spec((HID,)),
                      _full_spec((3, HID)), _full_spec((HID,)),
                      _full_spec((HID,)), _full_spec((HID, HEADS))],
            out_specs=[_row_spec(HID), _row_spec(HID), _row_spec(4),
                       _row_spec(DH), _row_spec(DH), _row_spec(DH),
                       _row_spec(DH)],
            out_shape=[jax.ShapeDtypeStruct((N, HID), jnp.float32),
                       jax.ShapeDtypeStruct((N, HID), jnp.float32),
                       jax.ShapeDtypeStruct((N, HEADS), jnp.float32),
                       jax.ShapeDtypeStruct((N, DH), jnp.float32),
                       jax.ShapeDtypeStruct((N, DH), jnp.float32),
                       jax.ShapeDtypeStruct((N, DH), jnp.float32),
                       jax.ShapeDtypeStruct((N, DH), jnp.float32)],
        )(h, dacc[0], dacc[1], p[pre + '_Wl'], p[pre + '_bl'],
          p[pre + '_Wr'], p[pre + '_br'], wip, bip, attf, sel)

        ex0, ex1, ex2, ex3, dn0, dn1, dn2, dn3 = _sc_logits(
            xl, xrf, src, dst, edge_attr, wipp, attf, z1)
        den4 = jnp.stack([dn0[0] + dn0[1], dn1[0] + dn1[1],
                          dn2[0] + dn2[1], dn3[0] + dn3[1]], axis=1)
        onum = _sc_scatter(x0, x1, x2, x3, src, dst,
                           ex0, ex1, ex2, ex3, z32[:, :DH])

        h = pl.pallas_call(
            _combine_kernel,
            grid=(GRID,),
            in_specs=[_row_spec(DH)] * 8 + [_row_spec(4),
                      _row_spec(4), _row_spec(HID), _row_spec(HID),
                      _full_spec((HID,)), _full_spec((HID,)),
                      _full_spec((HID,)), _full_spec((HEADS, HID))],
            out_specs=_row_spec(HID),
            out_shape=jax.ShapeDtypeStruct((N, HID), jnp.float32),
        )(*onum,
          den4, exs, xl, hres, p[pre + '_bias'],
          p['n%d_g' % i], p['n%d_b' % i], sel.T)

    gm = pl.pallas_call(
        _pool_kernel,
        grid=(GRID,),
        in_specs=[_row_spec(HID), pl.BlockSpec((R, 1), lambda i: (i, 0))],
        out_specs=_full_spec((B, HID)),
        out_shape=jax.ShapeDtypeStruct((B, HID), jnp.float32),
        scratch_shapes=[pltpu.VMEM((B, HID), jnp.float32),
                        pltpu.VMEM((B, HID), jnp.float32)],
    )(h, batch.reshape(N, 1))

    gfp = jnp.pad(global_features, ((0, 0), (0, 5)))
    gewp = jnp.pad(p['ge_W'], ((0, 5), (0, 0)))
    preds, unc = pl.pallas_call(
        _head_kernel,
        out_shape=(jax.ShapeDtypeStruct((B, 3), jnp.float32),
                   jax.ShapeDtypeStruct((B, 3), jnp.float32)),
    )(gm, gfp, gewp, p['ge_b'], p['ge_g'], p['ge_beta'],
      p['p1_W'][:HID], p['p1_W'][HID:], p['p1_b'], p['p2_W'], p['p2_b'],
      p['p3_W'], p['p3_b'],
      p['u1_W'][:HID], p['u1_W'][HID:], p['u1_b'], p['u2_W'], p['u2_b'])
    return preds, unc


# speedup vs baseline: 25.4795x; 1.0549x over previous
import functools

import jax
import jax.numpy as jnp
from jax import lax
from jax.experimental import pallas as pl
from jax.experimental.pallas import tpu as pltpu
from jax.experimental.pallas import tpu_sc as plsc

N = 50000
E = 800000
B = 8
HID = 128
HEADS = 4
DH = HID // HEADS

NC = 2
NS = 16
BLK = 320
EPC = E // NC
BPC = EPC // BLK
NBF = BPC // NS
NBR = BPC - NBF * NS
NPAD = 50048
NT = NPAD // NS
R = 1000
GRID = N // R

_SC_MESH = plsc.VectorSubcoreMesh(core_axis_name="c", subcore_axis_name="s")
_SC_PARAMS = pltpu.CompilerParams(needs_layout_passes=False,
                                  use_tc_tiling_on_sc=False)


def _pgelu(h):
    return 0.5 * h * (1.0 + jax.lax.erf(h * 0.7071067811865476))


def _softplus(h):
    return jnp.logaddexp(h, 0.0)


def _pln(h, g, b):
    m = jnp.mean(h, axis=-1, keepdims=True)
    v = jnp.mean((h - m) ** 2, axis=-1, keepdims=True)
    return (h - m) / jnp.sqrt(v + 1e-5) * g + b


def _dot(a, b):
    return jnp.dot(a, b, preferred_element_type=jnp.float32,
                   precision=jax.lax.Precision.HIGHEST)


def _tile_layout():
    c = lax.axis_index("c")
    s = lax.axis_index("s")
    nblk = jnp.where(s < NBR, NBF + 1, NBF)
    return c, s, nblk


def _blk_base(c, s, j):
    return c * EPC + (s + NS * j) * BLK


@functools.partial(
    pl.kernel,
    out_type=[jax.ShapeDtypeStruct((NC, NPAD, 32), jnp.float32)],
    scratch_types=[
        pltpu.VMEM((BLK,), jnp.int32),
        pltpu.VMEM((BLK, 3), jnp.float32),
        pltpu.VMEM((BLK, 32), jnp.float32),
        pltpu.VMEM_SHARED((NPAD, 32), jnp.float32),
        pltpu.SemaphoreType.DMA,
    ],
    mesh=_SC_MESH,
    compiler_params=_SC_PARAMS,
)
def _sc_deg(dst_hbm, ea_hbm, z32_hbm, dacc_hbm, dstv, eab, vb, accS, sem):
    c, s, nblk = _tile_layout()
    pltpu.sync_copy(z32_hbm, accS.at[pl.ds(s * NT, NT), :])
    iota = lax.iota(jnp.int32, 16)

    def zrow(r, carry):
        vb[r, pl.ds(0, 16)] = jnp.zeros((16,), jnp.float32)
        vb[r, pl.ds(16, 16)] = jnp.zeros((16,), jnp.float32)
        return carry

    lax.fori_loop(0, BLK, zrow, 0, unroll=4)
    for g in range(BLK // 16):
        plsc.store_scatter(vb, [g * 16 + iota, jnp.full((16,), 3, jnp.int32)],
                           jnp.full((16,), 1.0, jnp.float32))
    plsc.subcore_barrier()

    def body(j, carry):
        base = _blk_base(c, s, j)
        cp1 = pltpu.async_copy(dst_hbm.at[pl.ds(base, BLK)], dstv, sem)
        cp2 = pltpu.async_copy(ea_hbm.at[pl.ds(base, BLK), :], eab, sem)
        cp1.wait()
        cp2.wait()

        def group(g, carry2):
            eidx = g * 16 + iota
            for k in range(3):
                kv = jnp.full((16,), k, jnp.int32)
                plsc.store_scatter(vb, [eidx, kv],
                                   plsc.load_gather(eab, [eidx, kv]))
            return carry2

        lax.fori_loop(0, BLK // 16, group, 0)
        pltpu.sync_copy(vb, accS.at[dstv], add=True)
        return carry

    lax.fori_loop(0, nblk, body, 0)
    plsc.subcore_barrier()
    pltpu.sync_copy(accS.at[pl.ds(s * NT, NT), :],
                    dacc_hbm.at[c, pl.ds(s * NT, NT), :])


@functools.partial(
    pl.kernel,
    out_type=[
        jax.ShapeDtypeStruct((E,), jnp.float32),
        jax.ShapeDtypeStruct((E,), jnp.float32),
        jax.ShapeDtypeStruct((E,), jnp.float32),
        jax.ShapeDtypeStruct((E,), jnp.float32),
        jax.ShapeDtypeStruct((NC, NPAD), jnp.float32),
        jax.ShapeDtypeStruct((NC, NPAD), jnp.float32),
        jax.ShapeDtypeStruct((NC, NPAD), jnp.float32),
        jax.ShapeDtypeStruct((NC, NPAD), jnp.float32),
    ],
    scratch_types=[
        pltpu.VMEM((BLK,), jnp.int32),
        pltpu.VMEM((BLK,), jnp.int32),
        pltpu.VMEM((BLK, 3), jnp.float32),
        pltpu.VMEM((BLK, HID), jnp.float32),
        pltpu.VMEM((BLK, HID), jnp.float32),
        pltpu.VMEM((BLK,), jnp.float32),
        pltpu.VMEM((BLK,), jnp.float32),
        pltpu.VMEM((BLK,), jnp.float32),
        pltpu.VMEM((BLK,), jnp.float32),
        pltpu.VMEM((4, HID), jnp.float32),
        pltpu.VMEM((HID,), jnp.float32),
        pltpu.VMEM_SHARED((NPAD,), jnp.float32),
        pltpu.VMEM_SHARED((NPAD,), jnp.float32),
        pltpu.VMEM_SHARED((NPAD,), jnp.float32),
        pltpu.VMEM_SHARED((NPAD,), jnp.float32),
        pltpu.SemaphoreType.DMA,
        pltpu.SemaphoreType.DMA,
    ],
    mesh=_SC_MESH,
    compiler_params=_SC_PARAMS,
)
def _sc_logits(xl_hbm, xrf_hbm, src_hbm, dst_hbm, ea_hbm, wip_hbm, att_hbm,
               z1_hbm, ex0_hbm, ex1_hbm, ex2_hbm, ex3_hbm,
               den0_hbm, den1_hbm, den2_hbm, den3_hbm,
               srcv, dstv, eab, xlb, xrb, exb0, exb1, exb2, exb3,
               wipv, attb, acc0, acc1, acc2, acc3, sem, sem2):
    c, s, nblk = _tile_layout()
    accs = [acc0, acc1, acc2, acc3]
    dens = [den0_hbm, den1_hbm, den2_hbm, den3_hbm]
    exbs = [exb0, exb1, exb2, exb3]
    pltpu.sync_copy(wip_hbm, wipv)
    pltpu.sync_copy(att_hbm, attb)
    for a in accs:
        pltpu.sync_copy(z1_hbm, a.at[pl.ds(s * NT, NT)])
    plsc.subcore_barrier()
    iota = lax.iota(jnp.int32, 16)

    def body(j, carry):
        base = _blk_base(c, s, j)
        cp1 = pltpu.async_copy(src_hbm.at[pl.ds(base, BLK)], srcv, sem)
        cp2 = pltpu.async_copy(dst_hbm.at[pl.ds(base, BLK)], dstv, sem2)
        cp3 = pltpu.async_copy(ea_hbm.at[pl.ds(base, BLK), :], eab, sem)
        cp1.wait()
        cp2.wait()
        cp3.wait()
        cp4 = pltpu.async_copy(xl_hbm.at[srcv], xlb, sem)
        cp5 = pltpu.async_copy(xrf_hbm.at[dstv], xrb, sem2)
        cp4.wait()
        cp5.wait()

        def group(g, carry2):
            eidx = g * 16 + iota
            ea0 = plsc.load_gather(eab, [eidx, jnp.zeros((16,), jnp.int32)])
            ea1 = plsc.load_gather(eab, [eidx, jnp.full((16,), 1, jnp.int32)])
            ea2 = plsc.load_gather(eab, [eidx, jnp.full((16,), 2, jnp.int32)])
            zero16 = jnp.zeros((16,), jnp.int32)
            for h in range(HEADS):
                a_h = jnp.zeros((16,), jnp.float32)
                for cc in range(h * DH, (h + 1) * DH):
                    cv = jnp.full((16,), cc, jnp.int32)
                    w0 = plsc.load_gather(wipv, [zero16, cv])
                    w1 = plsc.load_gather(wipv, [zero16 + 1, cv])
                    w2 = plsc.load_gather(wipv, [zero16 + 2, cv])
                    av = plsc.load_gather(attb, [cv])
                    z = (plsc.load_gather(xlb, [eidx, cv])
                         + plsc.load_gather(xrb, [eidx, cv])
                         + ea0 * w0 + ea1 * w1 + ea2 * w2)
                    lz = jnp.maximum(z, 0.0) + 0.2 * jnp.minimum(z, 0.0)
                    a_h = a_h + lz * av
                ex_h = jnp.exp(a_h)
                [exb0, exb1, exb2, exb3][h][pl.ds(g * 16, 16)] = ex_h
            return carry2

        lax.fori_loop(0, BLK // 16, group, 0)
        for h, (exb, exh) in enumerate(
                zip(exbs, [ex0_hbm, ex1_hbm, ex2_hbm, ex3_hbm])):
            pltpu.sync_copy(exb, accs[h].at[dstv], add=True)
            pltpu.sync_copy(exb, exh.at[pl.ds(base, BLK)])
        return carry

    lax.fori_loop(0, nblk, body, 0)
    plsc.subcore_barrier()
    for h in range(HEADS):
        pltpu.sync_copy(accs[h].at[pl.ds(s * NT, NT)],
                        dens[h].at[c, pl.ds(s * NT, NT)])


@functools.partial(
    pl.kernel,
    out_type=[jax.ShapeDtypeStruct((NPAD, DH), jnp.float32)] * 8,
    scratch_types=[
        pltpu.VMEM((BLK,), jnp.int32),
        pltpu.VMEM((BLK,), jnp.int32),
        pltpu.VMEM((BLK,), jnp.float32),
        pltpu.VMEM((BLK, DH), jnp.float32),
        pltpu.VMEM((BLK, DH), jnp.float32),
        pltpu.VMEM_SHARED((NPAD, DH), jnp.float32),
        pltpu.SemaphoreType.DMA,
    ],
    mesh=_SC_MESH,
    compiler_params=_SC_PARAMS,
)
def _sc_scatter(xlh0_hbm, xlh1_hbm, xlh2_hbm, xlh3_hbm, src_hbm, dst_hbm,
                ex0_hbm, ex1_hbm, ex2_hbm, ex3_hbm, z32_hbm,
                o00, o01, o02, o03, o10, o11, o12, o13,
                srcv, dstv, wv, xb, ob, accS, sem):
    onum_out = [[o00, o01, o02, o03], [o10, o11, o12, o13]]
    c, s, nblk = _tile_layout()
    xlh = [xlh0_hbm, xlh1_hbm, xlh2_hbm, xlh3_hbm]
    exa = [ex0_hbm, ex1_hbm, ex2_hbm, ex3_hbm]
    for h in range(HEADS):
        pltpu.sync_copy(z32_hbm, accS.at[pl.ds(s * NT, NT), :])
        plsc.subcore_barrier()

        def body(j, carry):
            base = _blk_base(c, s, j)
            cp1 = pltpu.async_copy(src_hbm.at[pl.ds(base, BLK)], srcv, sem)
            cp2 = pltpu.async_copy(dst_hbm.at[pl.ds(base, BLK)], dstv, sem)
            cp3 = pltpu.async_copy(exa[h].at[pl.ds(base, BLK)], wv, sem)
            cp1.wait()
            cp2.wait()
            cp3.wait()
            pltpu.async_copy(xlh[h].at[srcv], xb, sem).wait()

            def edge(e, carry2):
                w = plsc.load_gather(wv, [jnp.full((16,), e, jnp.int32)])
                ob[e, pl.ds(0, 16)] = xb[e, pl.ds(0, 16)] * w
                ob[e, pl.ds(16, 16)] = xb[e, pl.ds(16, 16)] * w
                return carry2

            lax.fori_loop(0, BLK, edge, 0, unroll=4)
            pltpu.sync_copy(ob, accS.at[dstv], add=True)
            return carry

        lax.fori_loop(0, nblk, body, 0)
        plsc.subcore_barrier()

        @pl.when(c == 0)
        def _():
            pltpu.sync_copy(accS.at[pl.ds(s * NT, NT), :],
                            onum_out[0][h].at[pl.ds(s * NT, NT), :])

        @pl.when(c == 1)
        def _():
            pltpu.sync_copy(accS.at[pl.ds(s * NT, NT), :],
                            onum_out[1][h].at[pl.ds(s * NT, NT), :])

        plsc.subcore_barrier()


def _node_enc_kernel(x_ref, w_ref, b_ref, g_ref, beta_ref, o_ref):
    h = _dot(x_ref[...], w_ref[...]) + b_ref[...]
    o_ref[...] = _pgelu(_pln(h, g_ref[...], beta_ref[...]))


def _prep_kernel(h_ref, a0_ref, a1_ref, wl_ref, bl_ref, wr_ref, br_ref,
                 wip_ref, bip_ref, att_ref, sel_ref,
                 xl_ref, xrf_ref, exs_ref, x0_ref, x1_ref, x2_ref, x3_ref):
    h = h_ref[...]
    xl = _dot(h, wl_ref[...]) + bl_ref[...]
    xr = _dot(h, wr_ref[...]) + br_ref[...]
    bip = bip_ref[...]
    xl_ref[...] = xl
    xrf_ref[...] = xr + bip
    acc = a0_ref[...] + a1_ref[...]
    cnt = acc[:, 3:4]
    mc = jnp.maximum(cnt, 1.0)
    ea_mean = acc[:, 0:3] / mc
    scale = cnt / mc
    loop = _dot(ea_mean, wip_ref[...]) + scale * bip
    zs = xl + xr + loop
    lz = (jnp.maximum(zs, 0.0) + 0.2 * jnp.minimum(zs, 0.0)) * att_ref[...]
    exs_ref[...] = jnp.exp(_dot(lz, sel_ref[...]))
    x0_ref[...] = xl[:, 0:32]
    x1_ref[...] = xl[:, 32:64]
    x2_ref[...] = xl[:, 64:96]
    x3_ref[...] = xl[:, 96:128]


def _combine_kernel(p00, p01, p02, p03, p10, p11, p12, p13, den_ref,
                    exs_ref, xl_ref, hres_ref, bias_ref, g_ref, b_ref,
                    ex_ref, o_ref):
    exs = exs_ref[...]
    den = den_ref[...] + exs
    den128 = _dot(den, ex_ref[...])
    exs128 = _dot(exs, ex_ref[...])
    xl = xl_ref[...]
    num = jnp.concatenate(
        [p00[...] + p10[...], p01[...] + p11[...],
         p02[...] + p12[...], p03[...] + p13[...]], axis=-1)
    num = num + exs128 * xl
    out = num / den128 + bias_ref[...] + hres_ref[...]
    o_ref[...] = _pgelu(_pln(out, g_ref[...], b_ref[...]))


def _pool_kernel(h_ref, bat_ref, gm_ref, acc_ref, cnt_ref):
    i = pl.program_id(0)

    @pl.when(i == 0)
    def _():
        acc_ref[...] = jnp.zeros_like(acc_ref)
        cnt_ref[...] = jnp.zeros_like(cnt_ref)

    h = h_ref[...]
    bat = bat_ref[...]
    for b in range(B):
        mask = jnp.where(bat == b, 1.0, 0.0)
        acc_ref[b, :] += jnp.sum(mask * h, axis=0)
        cnt_ref[b, :] += jnp.sum(mask, axis=0) * jnp.ones((HID,), jnp.float32)

    @pl.when(i == GRID - 1)
    def _():
        gm_ref[...] = acc_ref[...] / jnp.maximum(cnt_ref[...], 1.0)


def _head_kernel(gm_ref, gfp_ref, gew_ref, geb_ref, geg_ref, gebeta_ref,
                 p1a, p1b_w, p1b, p2w, p2b, p3w, p3b,
                 u1a, u1b_w, u1b, u2w, u2b, preds_ref, unc_ref):
    gm = gm_ref[...]
    ge = _dot(gfp_ref[...], gew_ref[...]) + geb_ref[...]
    g = _pgelu(_pln(ge, geg_ref[...], gebeta_ref[...]))
    h1 = _pgelu(_dot(gm, p1a[...]) + _dot(g, p1b_w[...]) + p1b[...])
    h2 = _pgelu(_dot(h1, p2w[...]) + p2b[...])
    preds_ref[...] = _softplus(_dot(h2, p3w[...]) + p3b[...])
    u1 = _pgelu(_dot(gm, u1a[...]) + _dot(g, u1b_w[...]) + u1b[...])
    unc_ref[...] = _softplus(_dot(u1, u2w[...]) + u2b[...])


def _row_spec(cols):
    return pl.BlockSpec((R, cols), lambda i: (i, 0))


def _full_spec(shape):
    nd = len(shape)
    return pl.BlockSpec(shape, lambda i: (0,) * nd)


def kernel(x, edge_index, edge_attr, batch, global_features, params):
    p = params
    src, dst = edge_index[0], edge_index[1]
    z32 = jnp.zeros((NT, 32), jnp.float32)
    z1 = jnp.zeros((NT,), jnp.float32)
    sel = jnp.repeat(jnp.eye(HEADS, dtype=jnp.float32), DH, axis=0)

    h = pl.pallas_call(
        _node_enc_kernel,
        grid=(GRID,),
        in_specs=[_row_spec(4), _full_spec((4, HID)), _full_spec((HID,)),
                  _full_spec((HID,)), _full_spec((HID,))],
        out_specs=_row_spec(HID),
        out_shape=jax.ShapeDtypeStruct((N, HID), jnp.float32),
    )(x, p['ne_W'], p['ne_b'], p['ne_g'], p['ne_beta'])

    dacc = _sc_deg(dst, edge_attr, z32)[0]

    for i in range(1, 4):
        pre = 'c%d' % i
        wip = p['ee_W'] @ p[pre + '_We']
        bip = p['ee_b'] @ p[pre + '_We']
        wipp = jnp.concatenate([wip, jnp.zeros((1, HID), jnp.float32)], 0)
        attf = p[pre + '_att'].reshape(HID)
        hres = h

        xl, xrf, exs, x0, x1, x2, x3 = pl.pallas_call(
            _prep_kernel,
            grid=(GRID,),
            in_specs=[_row_spec(HID), _row_spec(32), _row_spec(32),
                      _full_spec((HID, HID)), _full_spec((HID,)),
                      _full_spec((HID, HID)), _full_spec((HID,)),
                      _full_spec((3, HID)), _full_spec((HID,)),
                      _full_spec((HID,)), _full_spec((HID, HEADS))],
            out_specs=[_row_spec(HID), _row_spec(HID), _row_spec(4),
                       _row_spec(DH), _row_spec(DH), _row_spec(DH),
                       _row_spec(DH)],
            out_shape=[jax.ShapeDtypeStruct((N, HID), jnp.float32),
                       jax.ShapeDtypeStruct((N, HID), jnp.float32),
                       jax.ShapeDtypeStruct((N, HEADS), jnp.float32),
                       jax.ShapeDtypeStruct((N, DH), jnp.float32),
                       jax.ShapeDtypeStruct((N, DH), jnp.float32),
                       jax.ShapeDtypeStruct((N, DH), jnp.float32),
                       jax.ShapeDtypeStruct((N, DH), jnp.float32)],
        )(h, dacc[0], dacc[1], p[pre + '_Wl'], p[pre + '_bl'],
          p[pre + '_Wr'], p[pre + '_br'], wip, bip, attf, sel)

        ex0, ex1, ex2, ex3, dn0, dn1, dn2, dn3 = _sc_logits(
            xl, xrf, src, dst, edge_attr, wipp, attf, z1)
        den4 = jnp.stack([dn0[0] + dn0[1], dn1[0] + dn1[1],
                          dn2[0] + dn2[1], dn3[0] + dn3[1]], axis=1)
        onum = _sc_scatter(x0, x1, x2, x3, src, dst,
                           ex0, ex1, ex2, ex3, z32[:, :DH])

        h = pl.pallas_call(
            _combine_kernel,
            grid=(GRID,),
            in_specs=[_row_spec(DH)] * 8 + [_row_spec(4),
                      _row_spec(4), _row_spec(HID), _row_spec(HID),
                      _full_spec((HID,)), _full_spec((HID,)),
                      _full_spec((HID,)), _full_spec((HEADS, HID))],
            out_specs=_row_spec(HID),
            out_shape=jax.ShapeDtypeStruct((N, HID), jnp.float32),
        )(*onum,
          den4, exs, xl, hres, p[pre + '_bias'],
          p['n%d_g' % i], p['n%d_b' % i], sel.T)

    gm = pl.pallas_call(
        _pool_kernel,
        grid=(GRID,),
        in_specs=[_row_spec(HID), pl.BlockSpec((R, 1), lambda i: (i, 0))],
        out_specs=_full_spec((B, HID)),
        out_shape=jax.ShapeDtypeStruct((B, HID), jnp.float32),
        scratch_shapes=[pltpu.VMEM((B, HID), jnp.float32),
                        pltpu.VMEM((B, HID), jnp.float32)],
    )(h, batch.reshape(N, 1))

    gfp = jnp.pad(global_features, ((0, 0), (0, 5)))
    gewp = jnp.pad(p['ge_W'], ((0, 5), (0, 0)))
    preds, unc = pl.pallas_call(
        _head_kernel,
        out_shape=(jax.ShapeDtypeStruct((B, 3), jnp.float32),
                   jax.ShapeDtypeStruct((B, 3), jnp.float32)),
    )(gm, gfp, gewp, p['ge_b'], p['ge_g'], p['ge_beta'],
      p['p1_W'][:HID], p['p1_W'][HID:], p['p1_b'], p['p2_W'], p['p2_b'],
      p['p3_W'], p['p3_b'],
      p['u1_W'][:HID], p['u1_W'][HID:], p['u1_b'], p['u2_W'], p['u2_b'])
    return preds, unc


# speedup vs baseline: 25.8136x; 1.0131x over previous
import functools

import jax
import jax.numpy as jnp
from jax import lax
from jax.experimental import pallas as pl
from jax.experimental.pallas import tpu as pltpu
from jax.experimental.pallas import tpu_sc as plsc

N = 50000
E = 800000
B = 8
HID = 128
HEADS = 4
DH = HID // HEADS

NC = 2
NS = 16
BLK = 320
EPC = E // NC
BPC = EPC // BLK
NBF = BPC // NS
NBR = BPC - NBF * NS
NPAD = 50048
NT = NPAD // NS
R = 1000
GRID = N // R

_SC_MESH = plsc.VectorSubcoreMesh(core_axis_name="c", subcore_axis_name="s")
_SC_PARAMS = pltpu.CompilerParams(needs_layout_passes=False,
                                  use_tc_tiling_on_sc=False)


def _pgelu(h):
    return 0.5 * h * (1.0 + jax.lax.erf(h * 0.7071067811865476))


def _softplus(h):
    return jnp.logaddexp(h, 0.0)


def _pln(h, g, b):
    m = jnp.mean(h, axis=-1, keepdims=True)
    v = jnp.mean((h - m) ** 2, axis=-1, keepdims=True)
    return (h - m) / jnp.sqrt(v + 1e-5) * g + b


def _dot(a, b):
    return jnp.dot(a, b, preferred_element_type=jnp.float32,
                   precision=jax.lax.Precision.HIGHEST)


def _tile_layout():
    c = lax.axis_index("c")
    s = lax.axis_index("s")
    nblk = jnp.where(s < NBR, NBF + 1, NBF)
    return c, s, nblk


def _blk_base(c, s, j):
    return c * EPC + (s + NS * j) * BLK


@functools.partial(
    pl.kernel,
    out_type=[jax.ShapeDtypeStruct((NC, NPAD, 32), jnp.float32)],
    scratch_types=[
        pltpu.VMEM((BLK,), jnp.int32),
        pltpu.VMEM((BLK, 3), jnp.float32),
        pltpu.VMEM((BLK, 32), jnp.float32),
        pltpu.VMEM_SHARED((NPAD, 32), jnp.float32),
        pltpu.SemaphoreType.DMA,
    ],
    mesh=_SC_MESH,
    compiler_params=_SC_PARAMS,
)
def _sc_deg(dst_hbm, ea_hbm, z32_hbm, dacc_hbm, dstv, eab, vb, accS, sem):
    c, s, nblk = _tile_layout()
    pltpu.sync_copy(z32_hbm, accS.at[pl.ds(s * NT, NT), :])
    iota = lax.iota(jnp.int32, 16)

    def zrow(r, carry):
        vb[r, pl.ds(0, 16)] = jnp.zeros((16,), jnp.float32)
        vb[r, pl.ds(16, 16)] = jnp.zeros((16,), jnp.float32)
        return carry

    lax.fori_loop(0, BLK, zrow, 0, unroll=4)
    for g in range(BLK // 16):
        plsc.store_scatter(vb, [g * 16 + iota, jnp.full((16,), 3, jnp.int32)],
                           jnp.full((16,), 1.0, jnp.float32))
    plsc.subcore_barrier()

    def body(j, carry):
        base = _blk_base(c, s, j)
        cp1 = pltpu.async_copy(dst_hbm.at[pl.ds(base, BLK)], dstv, sem)
        cp2 = pltpu.async_copy(ea_hbm.at[pl.ds(base, BLK), :], eab, sem)
        cp1.wait()
        cp2.wait()

        def group(g, carry2):
            eidx = g * 16 + iota
            for k in range(3):
                kv = jnp.full((16,), k, jnp.int32)
                plsc.store_scatter(vb, [eidx, kv],
                                   plsc.load_gather(eab, [eidx, kv]))
            return carry2

        lax.fori_loop(0, BLK // 16, group, 0)
        pltpu.sync_copy(vb, accS.at[dstv], add=True)
        return carry

    lax.fori_loop(0, nblk, body, 0)
    plsc.subcore_barrier()
    pltpu.sync_copy(accS.at[pl.ds(s * NT, NT), :],
                    dacc_hbm.at[c, pl.ds(s * NT, NT), :])


@functools.partial(
    pl.kernel,
    out_type=[
        jax.ShapeDtypeStruct((E,), jnp.float32),
        jax.ShapeDtypeStruct((E,), jnp.float32),
        jax.ShapeDtypeStruct((E,), jnp.float32),
        jax.ShapeDtypeStruct((E,), jnp.float32),
        jax.ShapeDtypeStruct((NC, NPAD), jnp.float32),
        jax.ShapeDtypeStruct((NC, NPAD), jnp.float32),
        jax.ShapeDtypeStruct((NC, NPAD), jnp.float32),
        jax.ShapeDtypeStruct((NC, NPAD), jnp.float32),
    ],
    scratch_types=[
        pltpu.VMEM((BLK,), jnp.int32),
        pltpu.VMEM((BLK,), jnp.int32),
        pltpu.VMEM((BLK, 3), jnp.float32),
        pltpu.VMEM((BLK, HID), jnp.float32),
        pltpu.VMEM((BLK, HID), jnp.float32),
        pltpu.VMEM((BLK,), jnp.float32),
        pltpu.VMEM((BLK,), jnp.float32),
        pltpu.VMEM((BLK,), jnp.float32),
        pltpu.VMEM((BLK,), jnp.float32),
        pltpu.VMEM((HEADS, BLK), jnp.float32),
        pltpu.VMEM((3, BLK), jnp.float32),
        pltpu.VMEM((4, HID), jnp.float32),
        pltpu.VMEM((HID,), jnp.float32),
        pltpu.VMEM_SHARED((NPAD,), jnp.float32),
        pltpu.VMEM_SHARED((NPAD,), jnp.float32),
        pltpu.VMEM_SHARED((NPAD,), jnp.float32),
        pltpu.VMEM_SHARED((NPAD,), jnp.float32),
        pltpu.SemaphoreType.DMA,
        pltpu.SemaphoreType.DMA,
    ],
    mesh=_SC_MESH,
    compiler_params=_SC_PARAMS,
)
def _sc_logits(xl_hbm, xrf_hbm, src_hbm, dst_hbm, ea_hbm, wip_hbm, att_hbm,
               z1_hbm, ex0_hbm, ex1_hbm, ex2_hbm, ex3_hbm,
               den0_hbm, den1_hbm, den2_hbm, den3_hbm,
               srcv, dstv, eab, xlb, xrb, exb0, exb1, exb2, exb3,
               abuf, eatb, wipv, attb, acc0, acc1, acc2, acc3, sem, sem2):
    c, s, nblk = _tile_layout()
    accs = [acc0, acc1, acc2, acc3]
    dens = [den0_hbm, den1_hbm, den2_hbm, den3_hbm]
    exbs = [exb0, exb1, exb2, exb3]
    pltpu.sync_copy(wip_hbm, wipv)
    pltpu.sync_copy(att_hbm, attb)
    for a in accs:
        pltpu.sync_copy(z1_hbm, a.at[pl.ds(s * NT, NT)])
    plsc.subcore_barrier()
    iota = lax.iota(jnp.int32, 16)

    def body(j, carry):
        base = _blk_base(c, s, j)
        cp1 = pltpu.async_copy(src_hbm.at[pl.ds(base, BLK)], srcv, sem)
        cp2 = pltpu.async_copy(dst_hbm.at[pl.ds(base, BLK)], dstv, sem2)
        cp3 = pltpu.async_copy(ea_hbm.at[pl.ds(base, BLK), :], eab, sem)
        cp1.wait()
        cp2.wait()
        cp3.wait()
        cp4 = pltpu.async_copy(xl_hbm.at[srcv], xlb, sem)
        cp5 = pltpu.async_copy(xrf_hbm.at[dstv], xrb, sem2)
        cp4.wait()
        cp5.wait()

        def egrp(g, carry2):
            eidx = g * 16 + iota
            for k in range(3):
                eatb[k, pl.ds(g * 16, 16)] = plsc.load_gather(
                    eab, [eidx, jnp.full((16,), k, jnp.int32)])
            for h in range(HEADS):
                abuf[h, pl.ds(g * 16, 16)] = jnp.zeros((16,), jnp.float32)
            return carry2

        lax.fori_loop(0, BLK // 16, egrp, 0)
        zero16 = jnp.zeros((16,), jnp.int32)

        def ctile(ct, carry2):
            c0 = ct * 4
            hh = ct // (DH // 4)
            ws = []
            for l in range(4):
                cv = jnp.full((16,), 1, jnp.int32) * (c0 + l)
                ws.append((cv,
                           plsc.load_gather(wipv, [zero16, cv]),
                           plsc.load_gather(wipv, [zero16 + 1, cv]),
                           plsc.load_gather(wipv, [zero16 + 2, cv]),
                           plsc.load_gather(attb, [cv])))

            def gb(g, carry3):
                eidx = g * 16 + iota
                ea0 = eatb[0, pl.ds(g * 16, 16)]
                ea1 = eatb[1, pl.ds(g * 16, 16)]
                ea2 = eatb[2, pl.ds(g * 16, 16)]
                acc = abuf[hh, pl.ds(g * 16, 16)]
                for cv, w0, w1, w2, av in ws:
                    z = (plsc.load_gather(xlb, [eidx, cv])
                         + plsc.load_gather(xrb, [eidx, cv])
                         + ea0 * w0 + ea1 * w1 + ea2 * w2)
                    lz = jnp.maximum(z, 0.0) + 0.2 * jnp.minimum(z, 0.0)
                    acc = acc + lz * av
                abuf[hh, pl.ds(g * 16, 16)] = acc
                return carry3

            lax.fori_loop(0, BLK // 16, gb, 0)
            return carry2

        lax.fori_loop(0, HID // 4, ctile, 0)

        def expg(g, carry2):
            for h in range(HEADS):
                [exb0, exb1, exb2, exb3][h][pl.ds(g * 16, 16)] = jnp.exp(
                    abuf[h, pl.ds(g * 16, 16)])
            return carry2

        lax.fori_loop(0, BLK // 16, expg, 0)
        for h, (exb, exh) in enumerate(
                zip(exbs, [ex0_hbm, ex1_hbm, ex2_hbm, ex3_hbm])):
            pltpu.sync_copy(exb, accs[h].at[dstv], add=True)
            pltpu.sync_copy(exb, exh.at[pl.ds(base, BLK)])
        return carry

    lax.fori_loop(0, nblk, body, 0)
    plsc.subcore_barrier()
    for h in range(HEADS):
        pltpu.sync_copy(accs[h].at[pl.ds(s * NT, NT)],
                        dens[h].at[c, pl.ds(s * NT, NT)])


@functools.partial(
    pl.kernel,
    out_type=[jax.ShapeDtypeStruct((NPAD, DH), jnp.float32)] * 8,
    scratch_types=[
        pltpu.VMEM((BLK,), jnp.int32),
        pltpu.VMEM((BLK,), jnp.int32),
        pltpu.VMEM((BLK,), jnp.float32),
        pltpu.VMEM((BLK, DH), jnp.float32),
        pltpu.VMEM((BLK, DH), jnp.float32),
        pltpu.VMEM_SHARED((NPAD, DH), jnp.float32),
        pltpu.SemaphoreType.DMA,
    ],
    mesh=_SC_MESH,
    compiler_params=_SC_PARAMS,
)
def _sc_scatter(xlh0_hbm, xlh1_hbm, xlh2_hbm, xlh3_hbm, src_hbm, dst_hbm,
                ex0_hbm, ex1_hbm, ex2_hbm, ex3_hbm, z32_hbm,
                o00, o01, o02, o03, o10, o11, o12, o13,
                srcv, dstv, wv, xb, ob, accS, sem):
    onum_out = [[o00, o01, o02, o03], [o10, o11, o12, o13]]
    c, s, nblk = _tile_layout()
    xlh = [xlh0_hbm, xlh1_hbm, xlh2_hbm, xlh3_hbm]
    exa = [ex0_hbm, ex1_hbm, ex2_hbm, ex3_hbm]
    for h in range(HEADS):
        pltpu.sync_copy(z32_hbm, accS.at[pl.ds(s * NT, NT), :])
        plsc.subcore_barrier()

        def body(j, carry):
            base = _blk_base(c, s, j)
            cp1 = pltpu.async_copy(src_hbm.at[pl.ds(base, BLK)], srcv, sem)
            cp2 = pltpu.async_copy(dst_hbm.at[pl.ds(base, BLK)], dstv, sem)
            cp3 = pltpu.async_copy(exa[h].at[pl.ds(base, BLK)], wv, sem)
            cp1.wait()
            cp2.wait()
            cp3.wait()
            pltpu.async_copy(xlh[h].at[srcv], xb, sem).wait()

            def edge(e, carry2):
                w = plsc.load_gather(wv, [jnp.full((16,), e, jnp.int32)])
                ob[e, pl.ds(0, 16)] = xb[e, pl.ds(0, 16)] * w
                ob[e, pl.ds(16, 16)] = xb[e, pl.ds(16, 16)] * w
                return carry2

            lax.fori_loop(0, BLK, edge, 0, unroll=4)
            pltpu.sync_copy(ob, accS.at[dstv], add=True)
            return carry

        lax.fori_loop(0, nblk, body, 0)
        plsc.subcore_barrier()

        @pl.when(c == 0)
        def _():
            pltpu.sync_copy(accS.at[pl.ds(s * NT, NT), :],
                            onum_out[0][h].at[pl.ds(s * NT, NT), :])

        @pl.when(c == 1)
        def _():
            pltpu.sync_copy(accS.at[pl.ds(s * NT, NT), :],
                            onum_out[1][h].at[pl.ds(s * NT, NT), :])

        plsc.subcore_barrier()


def _node_enc_kernel(x_ref, w_ref, b_ref, g_ref, beta_ref, o_ref):
    h = _dot(x_ref[...], w_ref[...]) + b_ref[...]
    o_ref[...] = _pgelu(_pln(h, g_ref[...], beta_ref[...]))


def _prep_kernel(h_ref, a0_ref, a1_ref, wl_ref, bl_ref, wr_ref, br_ref,
                 wip_ref, bip_ref, att_ref, sel_ref,
                 xl_ref, xrf_ref, exs_ref, x0_ref, x1_ref, x2_ref, x3_ref):
    h = h_ref[...]
    xl = _dot(h, wl_ref[...]) + bl_ref[...]
    xr = _dot(h, wr_ref[...]) + br_ref[...]
    bip = bip_ref[...]
    xl_ref[...] = xl
    xrf_ref[...] = xr + bip
    acc = a0_ref[...] + a1_ref[...]
    cnt = acc[:, 3:4]
    mc = jnp.maximum(cnt, 1.0)
    ea_mean = acc[:, 0:3] / mc
    scale = cnt / mc
    loop = _dot(ea_mean, wip_ref[...]) + scale * bip
    zs = xl + xr + loop
    lz = (jnp.maximum(zs, 0.0) + 0.2 * jnp.minimum(zs, 0.0)) * att_ref[...]
    exs_ref[...] = jnp.exp(_dot(lz, sel_ref[...]))
    x0_ref[...] = xl[:, 0:32]
    x1_ref[...] = xl[:, 32:64]
    x2_ref[...] = xl[:, 64:96]
    x3_ref[...] = xl[:, 96:128]


def _combine_kernel(p00, p01, p02, p03, p10, p11, p12, p13, den_ref,
                    exs_ref, xl_ref, hres_ref, bias_ref, g_ref, b_ref,
                    ex_ref, o_ref):
    exs = exs_ref[...]
    den = den_ref[...] + exs
    den128 = _dot(den, ex_ref[...])
    exs128 = _dot(exs, ex_ref[...])
    xl = xl_ref[...]
    num = jnp.concatenate(
        [p00[...] + p10[...], p01[...] + p11[...],
         p02[...] + p12[...], p03[...] + p13[...]], axis=-1)
    num = num + exs128 * xl
    out = num / den128 + bias_ref[...] + hres_ref[...]
    o_ref[...] = _pgelu(_pln(out, g_ref[...], b_ref[...]))


def _pool_kernel(h_ref, bat_ref, gm_ref, acc_ref, cnt_ref):
    i = pl.program_id(0)

    @pl.when(i == 0)
    def _():
        acc_ref[...] = jnp.zeros_like(acc_ref)
        cnt_ref[...] = jnp.zeros_like(cnt_ref)

    h = h_ref[...]
    bat = bat_ref[...]
    for b in range(B):
        mask = jnp.where(bat == b, 1.0, 0.0)
        acc_ref[b, :] += jnp.sum(mask * h, axis=0)
        cnt_ref[b, :] += jnp.sum(mask, axis=0) * jnp.ones((HID,), jnp.float32)

    @pl.when(i == GRID - 1)
    def _():
        gm_ref[...] = acc_ref[...] / jnp.maximum(cnt_ref[...], 1.0)


def _head_kernel(gm_ref, gfp_ref, gew_ref, geb_ref, geg_ref, gebeta_ref,
                 p1a, p1b_w, p1b, p2w, p2b, p3w, p3b,
                 u1a, u1b_w, u1b, u2w, u2b, preds_ref, unc_ref):
    gm = gm_ref[...]
    ge = _dot(gfp_ref[...], gew_ref[...]) + geb_ref[...]
    g = _pgelu(_pln(ge, geg_ref[...], gebeta_ref[...]))
    h1 = _pgelu(_dot(gm, p1a[...]) + _dot(g, p1b_w[...]) + p1b[...])
    h2 = _pgelu(_dot(h1, p2w[...]) + p2b[...])
    preds_ref[...] = _softplus(_dot(h2, p3w[...]) + p3b[...])
    u1 = _pgelu(_dot(gm, u1a[...]) + _dot(g, u1b_w[...]) + u1b[...])
    unc_ref[...] = _softplus(_dot(u1, u2w[...]) + u2b[...])


def _row_spec(cols):
    return pl.BlockSpec((R, cols), lambda i: (i, 0))


def _full_spec(shape):
    nd = len(shape)
    return pl.BlockSpec(shape, lambda i: (0,) * nd)


def kernel(x, edge_index, edge_attr, batch, global_features, params):
    p = params
    src, dst = edge_index[0], edge_index[1]
    z32 = jnp.zeros((NT, 32), jnp.float32)
    z1 = jnp.zeros((NT,), jnp.float32)
    sel = jnp.repeat(jnp.eye(HEADS, dtype=jnp.float32), DH, axis=0)

    h = pl.pallas_call(
        _node_enc_kernel,
        grid=(GRID,),
        in_specs=[_row_spec(4), _full_spec((4, HID)), _full_spec((HID,)),
                  _full_spec((HID,)), _full_spec((HID,))],
        out_specs=_row_spec(HID),
        out_shape=jax.ShapeDtypeStruct((N, HID), jnp.float32),
    )(x, p['ne_W'], p['ne_b'], p['ne_g'], p['ne_beta'])

    dacc = _sc_deg(dst, edge_attr, z32)[0]

    for i in range(1, 4):
        pre = 'c%d' % i
        wip = p['ee_W'] @ p[pre + '_We']
        bip = p['ee_b'] @ p[pre + '_We']
        wipp = jnp.concatenate([wip, jnp.zeros((1, HID), jnp.float32)], 0)
        attf = p[pre + '_att'].reshape(HID)
        hres = h

        xl, xrf, exs, x0, x1, x2, x3 = pl.pallas_call(
            _prep_kernel,
            grid=(GRID,),
            in_specs=[_row_spec(HID), _row_spec(32), _row_spec(32),
                      _full_spec((HID, HID)), _full_spec((HID,)),
                      _full_spec((HID, HID)), _full_spec((HID,)),
                      _full_spec((3, HID)), _full_spec((HID,)),
                      _full_spec((HID,)), _full_spec((HID, HEADS))],
            out_specs=[_row_spec(HID), _row_spec(HID), _row_spec(4),
                       _row_spec(DH), _row_spec(DH), _row_spec(DH),
                       _row_spec(DH)],
            out_shape=[jax.ShapeDtypeStruct((N, HID), jnp.float32),
                       jax.ShapeDtypeStruct((N, HID), jnp.float32),
                       jax.ShapeDtypeStruct((N, HEADS), jnp.float32),
                       jax.ShapeDtypeStruct((N, DH), jnp.float32),
                       jax.ShapeDtypeStruct((N, DH), jnp.float32),
                       jax.ShapeDtypeStruct((N, DH), jnp.float32),
                       jax.ShapeDtypeStruct((N, DH), jnp.float32)],
        )(h, dacc[0], dacc[1], p[pre + '_Wl'], p[pre + '_bl'],
          p[pre + '_Wr'], p[pre + '_br'], wip, bip, attf, sel)

        ex0, ex1, ex2, ex3, dn0, dn1, dn2, dn3 = _sc_logits(
            xl, xrf, src, dst, edge_attr, wipp, attf, z1)
        den4 = jnp.stack([dn0[0] + dn0[1], dn1[0] + dn1[1],
                          dn2[0] + dn2[1], dn3[0] + dn3[1]], axis=1)
        onum = _sc_scatter(x0, x1, x2, x3, src, dst,
                           ex0, ex1, ex2, ex3, z32[:, :DH])

        h = pl.pallas_call(
            _combine_kernel,
            grid=(GRID,),
            in_specs=[_row_spec(DH)] * 8 + [_row_spec(4),
                      _row_spec(4), _row_spec(HID), _row_spec(HID),
                      _full_spec((HID,)), _full_spec((HID,)),
                      _full_spec((HID,)), _full_spec((HEADS, HID))],
            out_specs=_row_spec(HID),
            out_shape=jax.ShapeDtypeStruct((N, HID), jnp.float32),
        )(*onum,
          den4, exs, xl, hres, p[pre + '_bias'],
          p['n%d_g' % i], p['n%d_b' % i], sel.T)

    gm = pl.pallas_call(
        _pool_kernel,
        grid=(GRID,),
        in_specs=[_row_spec(HID), pl.BlockSpec((R, 1), lambda i: (i, 0))],
        out_specs=_full_spec((B, HID)),
        out_shape=jax.ShapeDtypeStruct((B, HID), jnp.float32),
        scratch_shapes=[pltpu.VMEM((B, HID), jnp.float32),
                        pltpu.VMEM((B, HID), jnp.float32)],
    )(h, batch.reshape(N, 1))

    gfp = jnp.pad(global_features, ((0, 0), (0, 5)))
    gewp = jnp.pad(p['ge_W'], ((0, 5), (0, 0)))
    preds, unc = pl.pallas_call(
        _head_kernel,
        out_shape=(jax.ShapeDtypeStruct((B, 3), jnp.float32),
                   jax.ShapeDtypeStruct((B, 3), jnp.float32)),
    )(gm, gfp, gewp, p['ge_b'], p['ge_g'], p['ge_beta'],
      p['p1_W'][:HID], p['p1_W'][HID:], p['p1_b'], p['p2_W'], p['p2_b'],
      p['p3_W'], p['p3_b'],
      p['u1_W'][:HID], p['u1_W'][HID:], p['u1_b'], p['u2_W'], p['u2_b'])
    return preds, unc
